# baseline (speedup 1.0000x reference)
import jax
import jax.numpy as jnp
from jax import lax
from jax.experimental import pallas as pl
from jax.experimental.pallas import tpu as pltpu
from jax.experimental.pallas import tpu_sc as plsc

VOCAB = 1000000
MAXLEN = 200
EMBED_DIM = 32
BATCH = 4096
SEQ = 200

NC = 2
NS = 16
NW = NC * NS

BB = BATCH // NW
NBUF = 4
DH = EMBED_DIM // 8


def _sc_kernel(xT_hbm, tok_hbm, pos_hbm, out_hbm, idx_v, g_v, t_v, pos_v, *sems):
    sem_g = sems[:NBUF]
    sem_s = sems[NBUF:]
    wid = lax.axis_index("s") * NC + lax.axis_index("c")
    b0 = wid * BB

    pltpu.sync_copy(pos_hbm, pos_v)
    pltpu.sync_copy(xT_hbm.at[:, pl.ds(b0, BB)], idx_v)

    lanes = lax.iota(jnp.int32, 16)
    rows = [lanes + (16 * i) for i in range(BB // 16)]

    def fire_gather(s, b):
        pltpu.async_copy(tok_hbm.at[idx_v.at[s]], g_v.at[b], sem_g[b])

    def drain_gather(s, b):
        pltpu.make_async_copy(tok_hbm.at[idx_v.at[s]], g_v.at[b],
                              sem_g[b]).wait()

    def compute(s, b):
        ssplat = jnp.broadcast_to(s, (16,))

        def d_body(dd, carry):
            for k in range(4):
                d = dd * 4 + k
                dh = lax.shift_right_logical(d, 3)
                dl = lax.bitwise_and(d, 7)
                dcol = jnp.broadcast_to(d, (16,))
                pval = plsc.load_gather(pos_v, [ssplat, dcol])
                for i in range(BB // 16):
                    vals = plsc.load_gather(g_v.at[b], [rows[i], dcol])
                    t_v[b, dh, dl, pl.ds(i * 16, 16)] = vals + pval
            return carry
        lax.fori_loop(0, EMBED_DIM // 4, d_body, 0)

    def fire_out(s, b):
        pltpu.async_copy(t_v.at[b], out_hbm.at[s, :, wid], sem_s[b])

    def drain_out(s, b):
        pltpu.make_async_copy(t_v.at[b], out_hbm.at[s, :, wid],
                              sem_s[b]).wait()

    for s in range(NBUF - 1):
        fire_gather(s, s)

    def main_body(it, carry):
        for j in range(NBUF):
            s = it * NBUF + j
            bn = (j + NBUF - 1) % NBUF

            @pl.when(s >= 1)
            def _():
                drain_out(s - 1, bn)

            @pl.when(s < SEQ - (NBUF - 1))
            def _():
                fire_gather(s + NBUF - 1, bn)

            drain_gather(s, j)
            compute(s, j)
            fire_out(s, j)
        return carry
    lax.fori_loop(0, SEQ // NBUF, main_body, 0)

    drain_out(SEQ - 1, (SEQ - 1) % NBUF)


def kernel(x, token_table, pos_table):
    xT = jnp.swapaxes(x, 0, 1).astype(jnp.int32)
    mesh = plsc.VectorSubcoreMesh(core_axis_name="c", subcore_axis_name="s",
                                  num_cores=NC, num_subcores=NS)
    out5 = pl.kernel(
        _sc_kernel,
        out_type=jax.ShapeDtypeStruct((SEQ, DH, NW, 8, BB), jnp.float32),
        mesh=mesh,
        compiler_params=pltpu.CompilerParams(use_tc_tiling_on_sc=False,
                                             needs_layout_passes=False),
        scratch_types=[
            pltpu.VMEM((SEQ, BB), jnp.int32),
            pltpu.VMEM((NBUF, BB, EMBED_DIM), jnp.float32),
            pltpu.VMEM((NBUF, DH, 8, BB), jnp.float32),
            pltpu.VMEM((MAXLEN, EMBED_DIM), jnp.float32),
        ] + [pltpu.SemaphoreType.DMA] * (2 * NBUF),
    )(xT, token_table, pos_table)
    return out5.transpose(2, 4, 0, 1, 3).reshape(BATCH, SEQ, EMBED_DIM)


# speedup vs baseline: 1.0334x; 1.0334x over previous
import jax
import jax.numpy as jnp
from jax import lax
from jax.experimental import pallas as pl
from jax.experimental.pallas import tpu as pltpu
from jax.experimental.pallas import tpu_sc as plsc

VOCAB = 1000000
MAXLEN = 200
EMBED_DIM = 32
BATCH = 4096
SEQ = 200

NC = 2
NS = 16
NW = NC * NS

BB = BATCH // NW
NBUF = 8
LOOK = 4
DH = EMBED_DIM // 8


def _sc_kernel(xT_hbm, tok_hbm, pos_hbm, out_hbm, idx_v, g_v, t_v, pos_v, *sems):
    sem_g = sems[:NBUF]
    sem_s = sems[NBUF:]
    wid = lax.axis_index("s") * NC + lax.axis_index("c")
    b0 = wid * BB

    pltpu.sync_copy(pos_hbm, pos_v)
    pltpu.sync_copy(xT_hbm.at[:, pl.ds(b0, BB)], idx_v)

    lanes = lax.iota(jnp.int32, 16)
    rows = [lanes + (16 * i) for i in range(BB // 16)]

    def fire_gather(s, b):
        pltpu.async_copy(tok_hbm.at[idx_v.at[s]], g_v.at[b], sem_g[b])

    def drain_gather(s, b):
        pltpu.make_async_copy(tok_hbm.at[idx_v.at[s]], g_v.at[b],
                              sem_g[b]).wait()

    def compute(s, b):
        ssplat = jnp.broadcast_to(s, (16,))

        def d_body(dd, carry):
            for k in range(4):
                d = dd * 4 + k
                dh = lax.shift_right_logical(d, 3)
                dl = lax.bitwise_and(d, 7)
                dcol = jnp.broadcast_to(d, (16,))
                pval = plsc.load_gather(pos_v, [ssplat, dcol])
                for i in range(BB // 16):
                    vals = plsc.load_gather(g_v.at[b], [rows[i], dcol])
                    t_v[b, dh, dl, pl.ds(i * 16, 16)] = vals + pval
            return carry
        lax.fori_loop(0, EMBED_DIM // 4, d_body, 0)

    def fire_out(s, b):
        pltpu.async_copy(t_v.at[b], out_hbm.at[s, :, wid], sem_s[b])

    def drain_out(s, b):
        pltpu.make_async_copy(t_v.at[b], out_hbm.at[s, :, wid],
                              sem_s[b]).wait()

    for s in range(LOOK):
        fire_gather(s, s)

    def main_body(it, carry):
        for j in range(NBUF):
            s = it * NBUF + j
            bn = (j + LOOK) % NBUF

            @pl.when(s >= NBUF - LOOK)
            def _():
                drain_out(s - (NBUF - LOOK), bn)

            @pl.when(s < SEQ - LOOK)
            def _():
                fire_gather(s + LOOK, bn)

            drain_gather(s, j)
            compute(s, j)
            fire_out(s, j)
        return carry
    lax.fori_loop(0, SEQ // NBUF, main_body, 0)

    for s in range(SEQ - (NBUF - LOOK), SEQ):
        drain_out(s, s % NBUF)


def kernel(x, token_table, pos_table):
    xT = jnp.swapaxes(x, 0, 1).astype(jnp.int32)
    mesh = plsc.VectorSubcoreMesh(core_axis_name="c", subcore_axis_name="s",
                                  num_cores=NC, num_subcores=NS)
    out5 = pl.kernel(
        _sc_kernel,
        out_type=jax.ShapeDtypeStruct((SEQ, DH, NW, 8, BB), jnp.float32),
        mesh=mesh,
        compiler_params=pltpu.CompilerParams(use_tc_tiling_on_sc=False,
                                             needs_layout_passes=False),
        scratch_types=[
            pltpu.VMEM((SEQ, BB), jnp.int32),
            pltpu.VMEM((NBUF, BB, EMBED_DIM), jnp.float32),
            pltpu.VMEM((NBUF, DH, 8, BB), jnp.float32),
            pltpu.VMEM((MAXLEN, EMBED_DIM), jnp.float32),
        ] + [pltpu.SemaphoreType.DMA] * (2 * NBUF),
    )(xT, token_table, pos_table)
    return out5.transpose(2, 4, 0, 1, 3).reshape(BATCH, SEQ, EMBED_DIM)
